# Initial kernel scaffold; baseline (speedup 1.0000x reference)
import jax
import jax.numpy as jnp
from jax.experimental import pallas as pl


def kernel(x, pos_edge_index, neg_edge_index, head, Wp1, Wn1, ap1, an1, Wp2, Wn2, ap2, an2, W):
    raise NotImplementedError("write your pallas kernel here")


# speedup vs baseline: 7.6922x; 7.6922x over previous
import jax
import jax.numpy as jnp
from jax.experimental import pallas as pl

_C = 0.5
_BN = 2048
_BE = 4096


def _dual_mm_kernel(x_ref, wa_ref, wb_ref, oa_ref, ob_ref):
    x = x_ref[...]
    oa_ref[...] = jnp.dot(x, wa_ref[...], preferred_element_type=jnp.float32)
    ob_ref[...] = jnp.dot(x, wb_ref[...], preferred_element_type=jnp.float32)


def _dual_mm(x, Wa, Wb):
    n, d = x.shape
    h = Wa.shape[1]
    return pl.pallas_call(
        _dual_mm_kernel,
        grid=(pl.cdiv(n, _BN),),
        in_specs=[
            pl.BlockSpec((_BN, d), lambda i: (i, 0)),
            pl.BlockSpec((d, h), lambda i: (0, 0)),
            pl.BlockSpec((d, h), lambda i: (0, 0)),
        ],
        out_specs=[
            pl.BlockSpec((_BN, h), lambda i: (i, 0)),
            pl.BlockSpec((_BN, h), lambda i: (i, 0)),
        ],
        out_shape=[
            jax.ShapeDtypeStruct((n, h), jnp.float32),
            jax.ShapeDtypeStruct((n, h), jnp.float32),
        ],
    )(x, Wa, Wb)


def _logits_kernel(hs_ref, hd_ref, asrc_ref, adst_ref, e_ref):
    e = jnp.dot(hs_ref[...], asrc_ref[...], preferred_element_type=jnp.float32)
    e = e + jnp.dot(hd_ref[...], adst_ref[...], preferred_element_type=jnp.float32)
    e_ref[...] = jnp.where(e > 0, e, 0.2 * e)


def _exp_kernel(e_ref, m_ref, ex_ref):
    ex_ref[...] = jnp.exp(e_ref[...] - m_ref[...])


def _weight_kernel(hs_ref, ex_ref, se_ref, b_ref, o_ref):
    alpha = ex_ref[...] / (se_ref[...] + 1e-16)
    o_ref[...] = hs_ref[...] * jnp.dot(
        alpha, b_ref[...], preferred_element_type=jnp.float32)


def _attention_agg(h, edge_index, att, n_nodes):
    src = edge_index[0]
    dst = edge_index[1]
    E = src.shape[0]
    head = att.shape[0]
    hid = h.shape[1]
    dh = hid // head

    lane_head = jnp.arange(hid, dtype=jnp.int32) // dh
    mask = lane_head[:, None] == jnp.arange(head, dtype=jnp.int32)[None, :]
    A_src = jnp.where(mask, att[:, :dh].reshape(hid)[:, None], 0.0)
    A_dst = jnp.where(mask, att[:, dh:].reshape(hid)[:, None], 0.0)
    B = mask.T.astype(jnp.float32)

    hs = jnp.take(h, src, axis=0)
    hd = jnp.take(h, dst, axis=0)

    grid = (pl.cdiv(E, _BE),)
    e = pl.pallas_call(
        _logits_kernel,
        grid=grid,
        in_specs=[
            pl.BlockSpec((_BE, hid), lambda i: (i, 0)),
            pl.BlockSpec((_BE, hid), lambda i: (i, 0)),
            pl.BlockSpec((hid, head), lambda i: (0, 0)),
            pl.BlockSpec((hid, head), lambda i: (0, 0)),
        ],
        out_specs=pl.BlockSpec((_BE, head), lambda i: (i, 0)),
        out_shape=jax.ShapeDtypeStruct((E, head), jnp.float32),
    )(hs, hd, A_src, A_dst)

    m = jax.ops.segment_max(e, dst, num_segments=n_nodes)
    m = jnp.where(jnp.isfinite(m), m, 0.0)
    me = jnp.take(m, dst, axis=0)

    ex = pl.pallas_call(
        _exp_kernel,
        grid=grid,
        in_specs=[
            pl.BlockSpec((_BE, head), lambda i: (i, 0)),
            pl.BlockSpec((_BE, head), lambda i: (i, 0)),
        ],
        out_specs=pl.BlockSpec((_BE, head), lambda i: (i, 0)),
        out_shape=jax.ShapeDtypeStruct((E, head), jnp.float32),
    )(e, me)

    s = jax.ops.segment_sum(ex, dst, num_segments=n_nodes)
    se = jnp.take(s, dst, axis=0)

    weighted = pl.pallas_call(
        _weight_kernel,
        grid=grid,
        in_specs=[
            pl.BlockSpec((_BE, hid), lambda i: (i, 0)),
            pl.BlockSpec((_BE, head), lambda i: (i, 0)),
            pl.BlockSpec((_BE, head), lambda i: (i, 0)),
            pl.BlockSpec((head, hid), lambda i: (0, 0)),
        ],
        out_specs=pl.BlockSpec((_BE, hid), lambda i: (i, 0)),
        out_shape=jax.ShapeDtypeStruct((E, hid), jnp.float32),
    )(hs, ex, se, B)

    return jax.ops.segment_sum(weighted, dst, num_segments=n_nodes)


def _combine_mm_kernel(agg_ref, h_ref, w_ref, x_ref, h2_ref):
    t = jnp.tanh(agg_ref[...] + _C * h_ref[...])
    x_ref[...] = t
    h2_ref[...] = jnp.dot(t, w_ref[...], preferred_element_type=jnp.float32)


def _combine_mm(agg, h, W):
    n, hid = h.shape
    return pl.pallas_call(
        _combine_mm_kernel,
        grid=(pl.cdiv(n, _BN),),
        in_specs=[
            pl.BlockSpec((_BN, hid), lambda i: (i, 0)),
            pl.BlockSpec((_BN, hid), lambda i: (i, 0)),
            pl.BlockSpec((hid, hid), lambda i: (0, 0)),
        ],
        out_specs=[
            pl.BlockSpec((_BN, hid), lambda i: (i, 0)),
            pl.BlockSpec((_BN, hid), lambda i: (i, 0)),
        ],
        out_shape=[
            jax.ShapeDtypeStruct((n, hid), jnp.float32),
            jax.ShapeDtypeStruct((n, hid), jnp.float32),
        ],
    )(agg, h, W)


def _final_kernel(pa_ref, hp_ref, na_ref, hn_ref, w1_ref, w2_ref, z_ref):
    px = jnp.tanh(pa_ref[...] + _C * hp_ref[...])
    nx = jnp.tanh(na_ref[...] + _C * hn_ref[...])
    z = jnp.dot(px, w1_ref[...], preferred_element_type=jnp.float32)
    z = z + jnp.dot(nx, w2_ref[...], preferred_element_type=jnp.float32)
    z_ref[...] = jnp.tanh(z)


def kernel(x, pos_edge_index, neg_edge_index, head, Wp1, Wn1, ap1, an1,
           Wp2, Wn2, ap2, an2, W):
    n = x.shape[0]
    hid = Wp1.shape[1]
    out = W.shape[1]

    hp, hn = _dual_mm(x, Wp1, Wn1)
    pos_agg = _attention_agg(hp, pos_edge_index, ap1, n)
    neg_agg = _attention_agg(hn, neg_edge_index, an1, n)

    pos_x, hp2 = _combine_mm(pos_agg, hp, Wp2)
    neg_x, hn2 = _combine_mm(neg_agg, hn, Wn2)

    pos_agg2 = _attention_agg(hp2, pos_edge_index, ap2, n)
    neg_agg2 = _attention_agg(hn2, neg_edge_index, an2, n)

    W1 = W[:hid, :]
    W2 = W[hid:, :]
    z = pl.pallas_call(
        _final_kernel,
        grid=(pl.cdiv(n, _BN),),
        in_specs=[
            pl.BlockSpec((_BN, hid), lambda i: (i, 0)),
            pl.BlockSpec((_BN, hid), lambda i: (i, 0)),
            pl.BlockSpec((_BN, hid), lambda i: (i, 0)),
            pl.BlockSpec((_BN, hid), lambda i: (i, 0)),
            pl.BlockSpec((hid, out), lambda i: (0, 0)),
            pl.BlockSpec((hid, out), lambda i: (0, 0)),
        ],
        out_specs=pl.BlockSpec((_BN, out), lambda i: (i, 0)),
        out_shape=jax.ShapeDtypeStruct((n, out), jnp.float32),
    )(pos_agg2, hp2, neg_agg2, hn2, W1, W2)

    m = jnp.concatenate([pos_agg2, neg_agg2], axis=-1)
    return z, m
